# baseline (speedup 1.0000x reference)
import functools

import jax
import jax.numpy as jnp
from jax import lax
from jax.experimental import pallas as pl
from jax.experimental.pallas import tpu as pltpu
from jax.experimental.pallas import tpu_sc as plsc

NC, NS = 2, 16
NU = 50000
NN = 100000
EH = 800000
PADH = 35584
EPH = EH + PADH
ROWS2D = 2 * EPH // 128
CH_E = 1024
NCHUNK = EPH // NS // CH_E
NIR = CH_E // 128
RPT = EPH // NS // 128
D = 64
DH = 16
NP = D // DH
ACC_R = 50048
HB = 6400
BM = 1000
NBH = NU // BM


def _deg_kernel(row2d):
    mesh = plsc.VectorSubcoreMesh(core_axis_name="c", subcore_axis_name="s")
    NB = HB * 16
    SB = NB // NS

    @functools.partial(
        pl.kernel,
        out_type=jax.ShapeDtypeStruct((NC, NB), jnp.float32),
        mesh=mesh,
        scratch_types=[
            pltpu.VMEM((NB,), jnp.float32),
            pltpu.VMEM((8, 128), jnp.int32),
            pltpu.VMEM((SB,), jnp.float32),
            pltpu.VMEM((SB,), jnp.float32),
            pltpu.VMEM_SHARED((NS, SB), jnp.float32),
        ],
        compiler_params=pltpu.CompilerParams(
            needs_layout_passes=False, use_tc_tiling_on_sc=False),
    )
    def k(row_hbm, deg_hbm, hist, rowv, accv, tmpv, parts):
        c = lax.axis_index("c")
        s = lax.axis_index("s")
        zero16 = jnp.zeros((16,), jnp.float32)
        ones16 = jnp.ones((16,), jnp.float32)

        def zh(i, _):
            hist[pl.ds(16 * i, 16)] = zero16
            return 0
        lax.fori_loop(0, NB // 16, zh, 0)

        def chunk(i, _):
            b = c * (RPT * NS) + s * RPT + 8 * i
            pltpu.sync_copy(row_hbm.at[pl.ds(b, 8)], rowv)

            def vec(t, _):
                jj = t // 8
                u = t - 8 * jj
                idx = rowv[jj, pl.ds(16 * u, 16)]
                plsc.addupdate_scatter(hist, [idx], ones16)
                return 0
            lax.fori_loop(0, 64, vec, 0)
            return 0
        lax.fori_loop(0, NCHUNK, chunk, 0)

        base = s * SB

        def cp(i, _):
            accv[pl.ds(16 * i, 16)] = hist[pl.ds(base + 16 * i, 16)]
            return 0
        lax.fori_loop(0, SB // 16, cp, 0)
        for r in range(1, NS):
            pub = lax.rem(s + r, NS)
            pltpu.sync_copy(hist.at[pl.ds(pub * SB, SB)], parts.at[s])
            plsc.subcore_barrier()
            src = lax.rem(s - r + NS, NS)
            pltpu.sync_copy(parts.at[src], tmpv)

            def red(i, _):
                accv[pl.ds(16 * i, 16)] = (accv[pl.ds(16 * i, 16)]
                                           + tmpv[pl.ds(16 * i, 16)])
                return 0
            lax.fori_loop(0, SB // 16, red, 0)
            plsc.subcore_barrier()

        @pl.when((c == 0) & (s == NU // SB))
        def _():
            pos = NU - (NU // SB) * SB
            accv[pl.ds(pos, 16)] = (accv[pl.ds(pos, 16)]
                                    - jnp.full((16,), float(PADH // 16),
                                               jnp.float32))
        pltpu.sync_copy(accv, deg_hbm.at[c, pl.ds(base, SB)])

    return k(row2d)


def _spmm(fsp, row2d, col2d):
    mesh = plsc.VectorSubcoreMesh(core_axis_name="c", subcore_axis_name="s")

    @functools.partial(
        pl.kernel,
        out_type=jax.ShapeDtypeStruct((NP, NN, DH), jnp.float32),
        mesh=mesh,
        scratch_types=[
            pltpu.VMEM((NIR, 128), jnp.int32),
            pltpu.VMEM((NIR, 128), jnp.int32),
            pltpu.VMEM((NIR, 128), jnp.int32),
            pltpu.VMEM((NIR, 128), jnp.int32),
            pltpu.VMEM((NIR, 128), jnp.int32),
            pltpu.VMEM((NIR, 128), jnp.int32),
            pltpu.VMEM((CH_E, DH), jnp.float32),
            pltpu.VMEM((CH_E, DH), jnp.float32),
            pltpu.VMEM((CH_E, DH), jnp.float32),
            pltpu.VMEM((782, DH), jnp.float32),
            pltpu.VMEM_SHARED((ACC_R, DH), jnp.float32),
            pltpu.SemaphoreType.DMA,
            pltpu.SemaphoreType.DMA,
            pltpu.SemaphoreType.DMA,
            pltpu.SemaphoreType.DMA,
            pltpu.SemaphoreType.DMA,
            pltpu.SemaphoreType.DMA,
            pltpu.SemaphoreType.DMA,
            pltpu.SemaphoreType.DMA,
            pltpu.SemaphoreType.DMA,
        ],
        compiler_params=pltpu.CompilerParams(
            needs_layout_passes=False, use_tc_tiling_on_sc=False),
    )
    def k(fsp_hbm, row_hbm, col_hbm, out_hbm,
          r0, r1, r2, k0, k1, k2, g0, g1, g2, zbuf, acc,
          is0, is1, is2, gs0, gs1, gs2, ss0, ss1, ss2):
        c = lax.axis_index("c")
        s = lax.axis_index("s")
        off = c * NU
        base_rc = c * (RPT * NS) + s * RPT
        zero16 = jnp.zeros((16,), jnp.float32)

        def zb(i, _):
            zbuf[i, :] = zero16
            return 0
        lax.fori_loop(0, 782, zb, 0)

        rwb = (r0, r1, r2)
        clb = (k0, k1, k2)
        gab = (g0, g1, g2)
        isem = (is0, is1, is2)
        gsem = (gs0, gs1, gs2)
        ssem = (ss0, ss1, ss2)

        def idx_load(i, b):
            pltpu.async_copy(row_hbm.at[pl.ds(base_rc + NIR * i, NIR)],
                             rwb[b], isem[b])
            pltpu.async_copy(col_hbm.at[pl.ds(base_rc + NIR * i, NIR)],
                             clb[b], isem[b])

        def idx_wait(b):
            pltpu.make_async_copy(row_hbm.at[pl.ds(base_rc, NIR)],
                                  rwb[b], isem[b]).wait()
            pltpu.make_async_copy(col_hbm.at[pl.ds(base_rc, NIR)],
                                  clb[b], isem[b]).wait()

        def localize(b):
            rw = rwb[b]

            def loc(t, _):
                jj = t // 8
                u = t - 8 * jj
                rw[jj, pl.ds(16 * u, 16)] = rw[jj, pl.ds(16 * u, 16)] - off
                return 0
            lax.fori_loop(0, 8 * NIR, loc, 0)

        def pass_body(p, _):
            def gathers(b):
                for j in range(NIR):
                    pltpu.async_copy(fsp_hbm.at[p].at[clb[b].at[j]],
                                     gab[b].at[pl.ds(128 * j, 128)], gsem[b])

            def wait_gathers(b):
                for j in range(NIR):
                    pltpu.make_async_copy(
                        fsp_hbm.at[p].at[clb[b].at[j]],
                        gab[b].at[pl.ds(128 * j, 128)], gsem[b]).wait()

            def scatters(b):
                for j in range(NIR):
                    pltpu.async_copy(gab[b].at[pl.ds(128 * j, 128)],
                                     acc.at[rwb[b].at[j]], ssem[b],
                                     add=True)

            def wait_scatters(b):
                for j in range(NIR):
                    pltpu.make_async_copy(gab[b].at[pl.ds(128 * j, 128)],
                                          acc.at[rwb[b].at[j]],
                                          ssem[b]).wait()

            for q in range(4):
                pltpu.sync_copy(zbuf, acc.at[pl.ds(3128 * s + 782 * q, 782)])
            plsc.subcore_barrier()

            idx_load(0, 0)
            idx_load(1, 1)
            idx_load(2, 2)
            idx_wait(0)
            localize(0)
            gathers(0)
            idx_wait(1)
            localize(1)
            gathers(1)

            def step(i, b, prep_gather, prep_load):
                if prep_gather:
                    idx_wait((b + 2) % 3)
                    localize((b + 2) % 3)
                    gathers((b + 2) % 3)
                wait_gathers(b)
                scatters(b)
                wait_scatters(b)
                if prep_load:
                    idx_load(i + 3, b)

            def steady(t, _):
                for b in range(3):
                    step(3 * t + b, b, True, True)
                return 0
            lax.fori_loop(0, (NCHUNK - 3) // 3, steady, 0)

            step(NCHUNK - 3, 0, True, False)
            step(NCHUNK - 2, 1, False, False)
            step(NCHUNK - 1, 2, False, False)

            plsc.subcore_barrier()
            pltpu.sync_copy(
                acc.at[pl.ds(3120 * s, 3120)],
                out_hbm.at[p, pl.ds(NU * c + 3120 * s, 3120)])

            @pl.when(s == NS - 1)
            def _():
                pltpu.sync_copy(
                    acc.at[pl.ds(3120 * NS, 80)],
                    out_hbm.at[p, pl.ds(NU * c + 3120 * NS, 80)])
            plsc.subcore_barrier()
            return 0

        lax.fori_loop(0, NP, pass_body, 0)

    return k(fsp, row2d, col2d)


def _prep_half(emb, d0, d1, h, prev=None):
    def body(*refs):
        if prev is None:
            e_ref, d0_ref, d1_ref, dinv_ref, fsp_ref = refs
        else:
            e_ref, d0_ref, d1_ref, _di, _fi, dinv_ref, fsp_ref = refs
        deg = d0_ref[...] + d1_ref[...]
        dinv = lax.rsqrt(deg + 1e-7)
        fs = e_ref[...] * dinv
        dinv_ref[...] = dinv
        for q in range(NP):
            fsp_ref[q] = fs[:, DH * q:DH * (q + 1)]

    in_specs = [pl.BlockSpec((BM, D), lambda i: (i, 0)),
                pl.BlockSpec((BM, 1), lambda i, h=h: (h * NBH + i, 0)),
                pl.BlockSpec((BM, 1), lambda i, h=h: (h * NBH + i, 0))]
    args = [emb, d0, d1]
    aliases = {}
    if prev is not None:
        in_specs += [pl.BlockSpec(memory_space=pl.ANY),
                     pl.BlockSpec(memory_space=pl.ANY)]
        args += [prev[0], prev[1]]
        aliases = {3: 0, 4: 1}
    return pl.pallas_call(
        body,
        grid=(NBH,),
        in_specs=in_specs,
        out_specs=[pl.BlockSpec((BM, 1), lambda i, h=h: (h * NBH + i, 0)),
                   pl.BlockSpec((NP, BM, DH),
                                lambda i, h=h: (0, h * NBH + i, 0))],
        out_shape=[jax.ShapeDtypeStruct((NN, 1), jnp.float32),
                   jax.ShapeDtypeStruct((NP, NN, DH), jnp.float32)],
        input_output_aliases=aliases,
    )(*args)


def _dense_half(xh, f, dinv, w, bias, grid_k, h, layer, emb=None,
                fsp_prev=None):
    emit_fsp = layer == 0

    def body(*refs):
        x_ref, f_ref, dinv_ref, w_ref, b_ref = refs[:5]
        if emit_fsp:
            y_ref, fsp_ref = refs[-2:]
        else:
            e_ref, out_ref = refs[5], refs[-1]
        dv = dinv_ref[...]
        x = jnp.concatenate([x_ref[q] for q in range(NP)], axis=1) * dv
        ft = f_ref[...]
        inter = x * ft
        c1 = jnp.cos(inter)
        s1 = jnp.sin(inter)
        p2 = (jnp.dot(c1, w_ref[0], preferred_element_type=jnp.float32)
              + jnp.dot(s1, w_ref[grid_k], preferred_element_type=jnp.float32)
              + b_ref[...])
        cg, sg = c1, s1
        for g in range(1, grid_k):
            cg, sg = cg * c1 - sg * s1, sg * c1 + cg * s1
            p2 = (p2
                  + jnp.dot(cg, w_ref[g], preferred_element_type=jnp.float32)
                  + jnp.dot(sg, w_ref[grid_k + g],
                            preferred_element_type=jnp.float32))
        y = ft + x + p2
        y = jnp.where(y >= 0, y, 0.2 * y)
        nrm = jnp.sqrt(jnp.sum(y * y, axis=1, keepdims=True))
        y = y / jnp.maximum(nrm, 1e-12)
        if layer == 0:
            y_ref[...] = y
            ys = y * dv
            for q in range(NP):
                fsp_ref[q] = ys[:, DH * q:DH * (q + 1)]
        else:
            out_ref[...] = jnp.concatenate([e_ref[...], ft, y], axis=1)

    in_specs = [pl.BlockSpec((NP, BM, DH), lambda i, h=h: (0, h * NBH + i, 0)),
                pl.BlockSpec((BM, D), lambda i: (i, 0)),
                pl.BlockSpec((BM, 1), lambda i, h=h: (h * NBH + i, 0)),
                pl.BlockSpec((2 * grid_k, D, D), lambda i: (0, 0, 0)),
                pl.BlockSpec((1, D), lambda i: (0, 0))]
    args = [xh, f, dinv, w, bias]
    if layer == 0:
        out_specs = [pl.BlockSpec((BM, D), lambda i: (i, 0)),
                     pl.BlockSpec((NP, BM, DH),
                                  lambda i, h=h: (0, h * NBH + i, 0))]
        out_shape = [jax.ShapeDtypeStruct((NU, D), jnp.float32),
                     jax.ShapeDtypeStruct((NP, NN, DH), jnp.float32)]
        aliases = {}
        if fsp_prev is not None:
            in_specs.append(pl.BlockSpec(memory_space=pl.ANY))
            args.append(fsp_prev)
            aliases = {5: 1}
    else:
        out_specs = [pl.BlockSpec((BM, 3 * D), lambda i: (i, 0))]
        out_shape = [jax.ShapeDtypeStruct((NU, 3 * D), jnp.float32)]
        in_specs.insert(5, pl.BlockSpec((BM, D), lambda i: (i, 0)))
        args.insert(5, emb)
        aliases = {}
    return pl.pallas_call(
        body,
        grid=(NBH,),
        in_specs=in_specs,
        out_specs=out_specs,
        out_shape=out_shape,
        input_output_aliases=aliases,
    )(*args)


def _kan_weight(fc):
    wc = fc[0].transpose(2, 1, 0)
    ws = fc[1].transpose(2, 1, 0)
    return jnp.concatenate([wc, ws], axis=0)


def kernel(user_emb, item_emb, lap_indices, lap_values, fc0, b0, fc1, b1):
    grid_k = fc0.shape[-1]
    row = lap_indices[0].astype(jnp.int32)
    col = lap_indices[1].astype(jnp.int32)

    ar = (jnp.arange(PADH, dtype=jnp.int32) % 16)
    pc0 = jnp.full((PADH,), NU, jnp.int32)
    pc1 = jnp.zeros((PADH,), jnp.int32)
    row_p = jnp.concatenate([row[:EH], NU + ar, row[EH:], NN + ar])
    col_p = jnp.concatenate([col[:EH], pc0, col[EH:], pc1])
    row2d = row_p.reshape(ROWS2D, 128)
    col2d = col_p.reshape(ROWS2D, 128)

    w1 = _kan_weight(fc0)
    w2 = _kan_weight(fc1)

    deg_parts = _deg_kernel(row2d)
    d0 = deg_parts[0, :NN].reshape(NN, 1)
    d1 = deg_parts[1, :NN].reshape(NN, 1)

    dinv_u, fsp0_u = _prep_half(user_emb, d0, d1, 0)
    dinv, fsp0 = _prep_half(item_emb, d0, d1, 1, prev=(dinv_u, fsp0_u))

    xh1 = _spmm(fsp0, row2d, col2d)
    y1u, fsp1_u = _dense_half(xh1, user_emb, dinv, w1, b0, grid_k, 0, 0)
    y1i, fsp1 = _dense_half(xh1, item_emb, dinv, w1, b0, grid_k, 1, 0,
                            fsp_prev=fsp1_u)
    xh2 = _spmm(fsp1, row2d, col2d)
    u_out, = _dense_half(xh2, y1u, dinv, w2, b1, grid_k, 0, 1, emb=user_emb)
    i_out, = _dense_half(xh2, y1i, dinv, w2, b1, grid_k, 1, 1, emb=item_emb)
    return u_out, i_out


# speedup vs baseline: 1.0005x; 1.0005x over previous
import functools

import jax
import jax.numpy as jnp
from jax import lax
from jax.experimental import pallas as pl
from jax.experimental.pallas import tpu as pltpu
from jax.experimental.pallas import tpu_sc as plsc

NC, NS = 2, 16
NU = 50000
NN = 100000
EH = 800000
PADH = 35584
EPH = EH + PADH
ROWS2D = 2 * EPH // 128
CH_E = 1024
NCHUNK = EPH // NS // CH_E
NIR = CH_E // 128
RPT = EPH // NS // 128
D = 64
DH = 16
NP = D // DH
ACC_R = 50048
HB = 6400
BM = 1000
NBH = NU // BM


def _deg_kernel(row2d):
    mesh = plsc.VectorSubcoreMesh(core_axis_name="c", subcore_axis_name="s")
    NB = HB * 16
    SB = NB // NS

    @functools.partial(
        pl.kernel,
        out_type=jax.ShapeDtypeStruct((NC, NB), jnp.float32),
        mesh=mesh,
        scratch_types=[
            pltpu.VMEM((NB,), jnp.float32),
            pltpu.VMEM((8, 128), jnp.int32),
            pltpu.VMEM((SB,), jnp.float32),
            pltpu.VMEM((SB,), jnp.float32),
            pltpu.VMEM_SHARED((NS, SB), jnp.float32),
        ],
        compiler_params=pltpu.CompilerParams(
            needs_layout_passes=False, use_tc_tiling_on_sc=False),
    )
    def k(row_hbm, deg_hbm, hist, rowv, accv, tmpv, parts):
        c = lax.axis_index("c")
        s = lax.axis_index("s")
        zero16 = jnp.zeros((16,), jnp.float32)
        ones16 = jnp.ones((16,), jnp.float32)

        def zh(i, _):
            hist[pl.ds(16 * i, 16)] = zero16
            return 0
        lax.fori_loop(0, NB // 16, zh, 0)

        def chunk(i, _):
            b = c * (RPT * NS) + s * RPT + 8 * i
            pltpu.sync_copy(row_hbm.at[pl.ds(b, 8)], rowv)

            def vec(t, _):
                jj = t // 8
                u = t - 8 * jj
                idx = rowv[jj, pl.ds(16 * u, 16)]
                plsc.addupdate_scatter(hist, [idx], ones16)
                return 0
            lax.fori_loop(0, 64, vec, 0)
            return 0
        lax.fori_loop(0, NCHUNK, chunk, 0)

        base = s * SB

        def cp(i, _):
            accv[pl.ds(16 * i, 16)] = hist[pl.ds(base + 16 * i, 16)]
            return 0
        lax.fori_loop(0, SB // 16, cp, 0)
        for r in range(1, NS):
            pub = lax.rem(s + r, NS)
            pltpu.sync_copy(hist.at[pl.ds(pub * SB, SB)], parts.at[s])
            plsc.subcore_barrier()
            src = lax.rem(s - r + NS, NS)
            pltpu.sync_copy(parts.at[src], tmpv)

            def red(i, _):
                accv[pl.ds(16 * i, 16)] = (accv[pl.ds(16 * i, 16)]
                                           + tmpv[pl.ds(16 * i, 16)])
                return 0
            lax.fori_loop(0, SB // 16, red, 0)
            plsc.subcore_barrier()

        @pl.when((c == 0) & (s == NU // SB))
        def _():
            pos = NU - (NU // SB) * SB
            accv[pl.ds(pos, 16)] = (accv[pl.ds(pos, 16)]
                                    - jnp.full((16,), float(PADH // 16),
                                               jnp.float32))
        pltpu.sync_copy(accv, deg_hbm.at[c, pl.ds(base, SB)])

    return k(row2d)


def _spmm(fsp, row2d, col2d):
    mesh = plsc.VectorSubcoreMesh(core_axis_name="c", subcore_axis_name="s")

    @functools.partial(
        pl.kernel,
        out_type=jax.ShapeDtypeStruct((NP, NN, DH), jnp.float32),
        mesh=mesh,
        scratch_types=[
            pltpu.VMEM((NIR, 128), jnp.int32),
            pltpu.VMEM((NIR, 128), jnp.int32),
            pltpu.VMEM((NIR, 128), jnp.int32),
            pltpu.VMEM((NIR, 128), jnp.int32),
            pltpu.VMEM((NIR, 128), jnp.int32),
            pltpu.VMEM((NIR, 128), jnp.int32),
            pltpu.VMEM((CH_E, DH), jnp.float32),
            pltpu.VMEM((CH_E, DH), jnp.float32),
            pltpu.VMEM((CH_E, DH), jnp.float32),
            pltpu.VMEM((782, DH), jnp.float32),
            pltpu.VMEM_SHARED((ACC_R, DH), jnp.float32),
            pltpu.SemaphoreType.DMA,
            pltpu.SemaphoreType.DMA,
            pltpu.SemaphoreType.DMA,
            pltpu.SemaphoreType.DMA,
            pltpu.SemaphoreType.DMA,
            pltpu.SemaphoreType.DMA,
            pltpu.SemaphoreType.DMA,
            pltpu.SemaphoreType.DMA,
            pltpu.SemaphoreType.DMA,
        ],
        compiler_params=pltpu.CompilerParams(
            needs_layout_passes=False, use_tc_tiling_on_sc=False),
    )
    def k(fsp_hbm, row_hbm, col_hbm, out_hbm,
          r0, r1, r2, k0, k1, k2, g0, g1, g2, zbuf, acc,
          is0, is1, is2, gs0, gs1, gs2, ss0, ss1, ss2):
        c = lax.axis_index("c")
        s = lax.axis_index("s")
        off = c * NU
        base_rc = c * (RPT * NS) + s * RPT
        zero16 = jnp.zeros((16,), jnp.float32)

        def zb(i, _):
            for u in range(DH // 16):
                zbuf[i, pl.ds(16 * u, 16)] = zero16
            return 0
        lax.fori_loop(0, 782, zb, 0)

        rwb = (r0, r1, r2)
        clb = (k0, k1, k2)
        gab = (g0, g1, g2)
        isem = (is0, is1, is2)
        gsem = (gs0, gs1, gs2)
        ssem = (ss0, ss1, ss2)

        def idx_load(i, b):
            pltpu.async_copy(row_hbm.at[pl.ds(base_rc + NIR * i, NIR)],
                             rwb[b], isem[b])
            pltpu.async_copy(col_hbm.at[pl.ds(base_rc + NIR * i, NIR)],
                             clb[b], isem[b])

        def idx_wait(b):
            pltpu.make_async_copy(row_hbm.at[pl.ds(base_rc, NIR)],
                                  rwb[b], isem[b]).wait()
            pltpu.make_async_copy(col_hbm.at[pl.ds(base_rc, NIR)],
                                  clb[b], isem[b]).wait()

        def localize(b):
            rw = rwb[b]

            def loc(t, _):
                jj = t // 8
                u = t - 8 * jj
                rw[jj, pl.ds(16 * u, 16)] = rw[jj, pl.ds(16 * u, 16)] - off
                return 0
            lax.fori_loop(0, 8 * NIR, loc, 0)

        def pass_body(p, _):
            def gathers(b):
                for j in range(NIR):
                    pltpu.async_copy(fsp_hbm.at[p].at[clb[b].at[j]],
                                     gab[b].at[pl.ds(128 * j, 128)], gsem[b])

            def wait_gathers(b):
                for j in range(NIR):
                    pltpu.make_async_copy(
                        fsp_hbm.at[p].at[clb[b].at[j]],
                        gab[b].at[pl.ds(128 * j, 128)], gsem[b]).wait()

            def scatters(b):
                for j in range(NIR):
                    pltpu.async_copy(gab[b].at[pl.ds(128 * j, 128)],
                                     acc.at[rwb[b].at[j]], ssem[b],
                                     add=True)

            def wait_scatters(b):
                for j in range(NIR):
                    pltpu.make_async_copy(gab[b].at[pl.ds(128 * j, 128)],
                                          acc.at[rwb[b].at[j]],
                                          ssem[b]).wait()

            for q in range(4):
                pltpu.sync_copy(zbuf, acc.at[pl.ds(3128 * s + 782 * q, 782)])
            plsc.subcore_barrier()

            idx_load(0, 0)
            idx_load(1, 1)
            idx_load(2, 2)
            idx_wait(0)
            localize(0)
            gathers(0)
            idx_wait(1)
            localize(1)
            gathers(1)

            def step(i, b, prep_gather, prep_load):
                if prep_gather:
                    idx_wait((b + 2) % 3)
                    localize((b + 2) % 3)
                    gathers((b + 2) % 3)
                wait_gathers(b)
                scatters(b)
                wait_scatters(b)
                if prep_load:
                    idx_load(i + 3, b)

            def steady(t, _):
                for b in range(3):
                    step(3 * t + b, b, True, True)
                return 0
            lax.fori_loop(0, (NCHUNK - 3) // 3, steady, 0)

            step(NCHUNK - 3, 0, True, False)
            step(NCHUNK - 2, 1, False, False)
            step(NCHUNK - 1, 2, False, False)

            plsc.subcore_barrier()
            pltpu.sync_copy(
                acc.at[pl.ds(3120 * s, 3120)],
                out_hbm.at[p, pl.ds(NU * c + 3120 * s, 3120)])

            @pl.when(s == NS - 1)
            def _():
                pltpu.sync_copy(
                    acc.at[pl.ds(3120 * NS, 80)],
                    out_hbm.at[p, pl.ds(NU * c + 3120 * NS, 80)])
            plsc.subcore_barrier()
            return 0

        lax.fori_loop(0, NP, pass_body, 0)

    return k(fsp, row2d, col2d)


def _prep_half(emb, d0, d1, h, prev=None):
    def body(*refs):
        if prev is None:
            e_ref, d0_ref, d1_ref, dinv_ref, fsp_ref = refs
        else:
            e_ref, d0_ref, d1_ref, _di, _fi, dinv_ref, fsp_ref = refs
        deg = d0_ref[...] + d1_ref[...]
        dinv = lax.rsqrt(deg + 1e-7)
        fs = e_ref[...] * dinv
        dinv_ref[...] = dinv
        for q in range(NP):
            fsp_ref[q] = fs[:, DH * q:DH * (q + 1)]

    in_specs = [pl.BlockSpec((BM, D), lambda i: (i, 0)),
                pl.BlockSpec((BM, 1), lambda i, h=h: (h * NBH + i, 0)),
                pl.BlockSpec((BM, 1), lambda i, h=h: (h * NBH + i, 0))]
    args = [emb, d0, d1]
    aliases = {}
    if prev is not None:
        in_specs += [pl.BlockSpec(memory_space=pl.ANY),
                     pl.BlockSpec(memory_space=pl.ANY)]
        args += [prev[0], prev[1]]
        aliases = {3: 0, 4: 1}
    return pl.pallas_call(
        body,
        grid=(NBH,),
        in_specs=in_specs,
        out_specs=[pl.BlockSpec((BM, 1), lambda i, h=h: (h * NBH + i, 0)),
                   pl.BlockSpec((NP, BM, DH),
                                lambda i, h=h: (0, h * NBH + i, 0))],
        out_shape=[jax.ShapeDtypeStruct((NN, 1), jnp.float32),
                   jax.ShapeDtypeStruct((NP, NN, DH), jnp.float32)],
        input_output_aliases=aliases,
    )(*args)


def _dense_half(xh, f, dinv, w, bias, grid_k, h, layer, emb=None,
                fsp_prev=None):
    emit_fsp = layer == 0

    def body(*refs):
        x_ref, f_ref, dinv_ref, w_ref, b_ref = refs[:5]
        if emit_fsp:
            y_ref, fsp_ref = refs[-2:]
        else:
            e_ref, out_ref = refs[5], refs[-1]
        dv = dinv_ref[...]
        x = jnp.concatenate([x_ref[q] for q in range(NP)], axis=1) * dv
        ft = f_ref[...]
        inter = x * ft
        c1 = jnp.cos(inter)
        s1 = jnp.sin(inter)
        p2 = (jnp.dot(c1, w_ref[0], preferred_element_type=jnp.float32)
              + jnp.dot(s1, w_ref[grid_k], preferred_element_type=jnp.float32)
              + b_ref[...])
        cg, sg = c1, s1
        for g in range(1, grid_k):
            cg, sg = cg * c1 - sg * s1, sg * c1 + cg * s1
            p2 = (p2
                  + jnp.dot(cg, w_ref[g], preferred_element_type=jnp.float32)
                  + jnp.dot(sg, w_ref[grid_k + g],
                            preferred_element_type=jnp.float32))
        y = ft + x + p2
        y = jnp.where(y >= 0, y, 0.2 * y)
        nrm = jnp.sqrt(jnp.sum(y * y, axis=1, keepdims=True))
        y = y / jnp.maximum(nrm, 1e-12)
        if layer == 0:
            y_ref[...] = y
            ys = y * dv
            for q in range(NP):
                fsp_ref[q] = ys[:, DH * q:DH * (q + 1)]
        else:
            out_ref[...] = jnp.concatenate([e_ref[...], ft, y], axis=1)

    in_specs = [pl.BlockSpec((NP, BM, DH), lambda i, h=h: (0, h * NBH + i, 0)),
                pl.BlockSpec((BM, D), lambda i: (i, 0)),
                pl.BlockSpec((BM, 1), lambda i, h=h: (h * NBH + i, 0)),
                pl.BlockSpec((2 * grid_k, D, D), lambda i: (0, 0, 0)),
                pl.BlockSpec((1, D), lambda i: (0, 0))]
    args = [xh, f, dinv, w, bias]
    if layer == 0:
        out_specs = [pl.BlockSpec((BM, D), lambda i: (i, 0)),
                     pl.BlockSpec((NP, BM, DH),
                                  lambda i, h=h: (0, h * NBH + i, 0))]
        out_shape = [jax.ShapeDtypeStruct((NU, D), jnp.float32),
                     jax.ShapeDtypeStruct((NP, NN, DH), jnp.float32)]
        aliases = {}
        if fsp_prev is not None:
            in_specs.append(pl.BlockSpec(memory_space=pl.ANY))
            args.append(fsp_prev)
            aliases = {5: 1}
    else:
        out_specs = [pl.BlockSpec((BM, 3 * D), lambda i: (i, 0))]
        out_shape = [jax.ShapeDtypeStruct((NU, 3 * D), jnp.float32)]
        in_specs.insert(5, pl.BlockSpec((BM, D), lambda i: (i, 0)))
        args.insert(5, emb)
        aliases = {}
    return pl.pallas_call(
        body,
        grid=(NBH,),
        in_specs=in_specs,
        out_specs=out_specs,
        out_shape=out_shape,
        input_output_aliases=aliases,
    )(*args)


def _kan_weight(fc):
    wc = fc[0].transpose(2, 1, 0)
    ws = fc[1].transpose(2, 1, 0)
    return jnp.concatenate([wc, ws], axis=0)


def kernel(user_emb, item_emb, lap_indices, lap_values, fc0, b0, fc1, b1):
    grid_k = fc0.shape[-1]
    row = lap_indices[0].astype(jnp.int32)
    col = lap_indices[1].astype(jnp.int32)

    ar = (jnp.arange(PADH, dtype=jnp.int32) % 16)
    pc0 = jnp.full((PADH,), NU, jnp.int32)
    pc1 = jnp.zeros((PADH,), jnp.int32)
    row_p = jnp.concatenate([row[:EH], NU + ar, row[EH:], NN + ar])
    col_p = jnp.concatenate([col[:EH], pc0, col[EH:], pc1])
    row2d = row_p.reshape(ROWS2D, 128)
    col2d = col_p.reshape(ROWS2D, 128)

    w1 = _kan_weight(fc0)
    w2 = _kan_weight(fc1)

    deg_parts = _deg_kernel(row2d)
    d0 = deg_parts[0, :NN].reshape(NN, 1)
    d1 = deg_parts[1, :NN].reshape(NN, 1)

    dinv_u, fsp0_u = _prep_half(user_emb, d0, d1, 0)
    dinv, fsp0 = _prep_half(item_emb, d0, d1, 1, prev=(dinv_u, fsp0_u))

    xh1 = _spmm(fsp0, row2d, col2d)
    y1u, fsp1_u = _dense_half(xh1, user_emb, dinv, w1, b0, grid_k, 0, 0)
    y1i, fsp1 = _dense_half(xh1, item_emb, dinv, w1, b0, grid_k, 1, 0,
                            fsp_prev=fsp1_u)
    xh2 = _spmm(fsp1, row2d, col2d)
    u_out, = _dense_half(xh2, y1u, dinv, w2, b1, grid_k, 0, 1, emb=user_emb)
    i_out, = _dense_half(xh2, y1i, dinv, w2, b1, grid_k, 1, 1, emb=item_emb)
    return u_out, i_out
